# Initial kernel scaffold; baseline (speedup 1.0000x reference)
import jax
import jax.numpy as jnp
from jax.experimental import pallas as pl


def kernel(graph_embedding, edge_index, weight):
    raise NotImplementedError("write your pallas kernel here")


# speedup vs baseline: 3.8110x; 3.8110x over previous
import functools

import jax
import jax.numpy as jnp
from jax import lax
from jax.experimental import pallas as pl
from jax.experimental.pallas import tpu as pltpu
from jax.experimental.pallas import tpu_sc as plsc

N = 10000
E = 320000
D = 128
DH = 64
NC = 2
NS = 16
CH = 128
IB = 8
NBLK = 10
CHUNKS = IB * NBLK
E_PAD = CHUNKS * NC * NS * CH
N_PAD = 10240
ZR = N_PAD // NS
DR = N_PAD // D
TBLK = 1024


def _sc_body(emb_hbm, src_hbm, dst_hbm, w_hbm, acc_out, prm_out,
             acc, deg, hist, sidx, didx, gbuf, pstage, dbuf, wv, iota, sem):
    c = lax.axis_index("c")
    s = lax.axis_index("s")
    r0 = s * ZR

    zf = jnp.zeros((16,), jnp.float32)

    def zrow(r, carry):
        for k in range(D // 16):
            gbuf[r, pl.ds(k * 16, 16)] = zf
        return carry

    lax.fori_loop(0, CH, zrow, 0)

    def zh(i, carry):
        for k in range(D // 16):
            hist[i, pl.ds(k * 16, 16)] = zf
        return carry

    lax.fori_loop(0, DR, zh, 0)

    for b in range(ZR // CH):
        pltpu.sync_copy(gbuf, acc.at[pl.ds(r0 + b * CH, CH), :])

    @pl.when(s == 0)
    def _():
        pltpu.sync_copy(gbuf.at[pl.ds(0, DR), :], deg)

    it16 = lax.iota(jnp.int32, 16)
    for k in range(DR // 16):
        iota[0, pl.ds(k * 16, 16)] = it16 + 16 * k

    pltpu.sync_copy(w_hbm.at[c], wv)

    plsc.subcore_barrier()

    ones16 = jnp.ones((16,), jnp.float32)

    def blk(b, carry):
        pltpu.sync_copy(src_hbm.at[c, s, pl.ds(b * IB, IB), :], sidx)
        pltpu.sync_copy(dst_hbm.at[c, s, pl.ds(b * IB, IB), :], didx)

        def chunk(j, carry2):
            pltpu.async_copy(emb_hbm.at[sidx.at[j]], gbuf, sem).wait()
            pltpu.sync_copy(gbuf, acc.at[didx.at[j]], add=True)
            return carry2

        lax.fori_loop(0, IB, chunk, 0)

        def hchunk(j, carry2):
            for k in range(CH // 16):
                idx = didx[j, pl.ds(k * 16, 16)]
                plsc.addupdate_scatter(
                    hist,
                    [lax.shift_right_logical(idx, 7),
                     lax.bitwise_and(idx, D - 1)],
                    ones16)
            return carry2

        return lax.fori_loop(0, IB, hchunk, carry)

    lax.fori_loop(0, NBLK, blk, 0)

    def blk2(b, carry):
        pltpu.sync_copy(dst_hbm.at[1 - c, s, pl.ds(b * IB, IB), :], didx)

        def hchunk(j, carry2):
            for k in range(CH // 16):
                idx = didx[j, pl.ds(k * 16, 16)]
                plsc.addupdate_scatter(
                    hist,
                    [lax.shift_right_logical(idx, 7),
                     lax.bitwise_and(idx, D - 1)],
                    ones16)
            return carry2

        return lax.fori_loop(0, IB, hchunk, carry)

    lax.fori_loop(0, NBLK, blk2, 0)

    pltpu.sync_copy(hist, deg.at[iota.at[0]], add=True)

    plsc.subcore_barrier()

    for b in range(ZR // CH):
        pltpu.sync_copy(acc.at[pl.ds(r0 + b * CH, CH), :], gbuf)
        pltpu.sync_copy(gbuf, acc_out.at[c, pl.ds(r0 + b * CH, CH), :])

    pltpu.sync_copy(deg.at[pl.ds(s * (ZR // D), ZR // D), :], dbuf)
    wvecs = [wv[0, pl.ds(k * 16, 16)] for k in range(DH // 16)]

    for bb in range(ZR // CH):
        def prow(j, carry, bb=bb):
            row = bb * CH + j
            dl = plsc.load_gather(
                dbuf, [jnp.full((16,), row // D, jnp.int32),
                       jnp.full((16,), row % D, jnp.int32)])
            for k in range(DH // 16):
                pstage[j, pl.ds(k * 16, 16)] = dl * wvecs[k]
            return carry

        lax.fori_loop(0, CH, prow, 0)
        pltpu.sync_copy(pstage, prm_out.at[c, pl.ds(r0 + bb * CH, CH), :])


_sc_call = pl.kernel(
    _sc_body,
    out_type=(
        jax.ShapeDtypeStruct((NC, N_PAD, D), jnp.float32),
        jax.ShapeDtypeStruct((NC, N_PAD, DH), jnp.float32),
    ),
    mesh=plsc.VectorSubcoreMesh(core_axis_name="c", subcore_axis_name="s"),
    compiler_params=pltpu.CompilerParams(needs_layout_passes=False),
    scratch_types=[
        pltpu.VMEM_SHARED((N_PAD, D), jnp.float32),
        pltpu.VMEM_SHARED((DR, D), jnp.float32),
        pltpu.VMEM((DR, D), jnp.float32),
        pltpu.VMEM((IB, CH), jnp.int32),
        pltpu.VMEM((IB, CH), jnp.int32),
        pltpu.VMEM((CH, D), jnp.float32),
        pltpu.VMEM((CH, DH), jnp.float32),
        pltpu.VMEM((ZR // D, D), jnp.float32),
        pltpu.VMEM((1, DH), jnp.float32),
        pltpu.VMEM((1, DR), jnp.int32),
        pltpu.SemaphoreType.DMA,
    ],
)


def _tc_body(acc_ref, prm_ref, out_ref):
    out_ref[:, :D] = acc_ref[0] + acc_ref[1]
    out_ref[:, D:D + DH] = prm_ref[0]
    out_ref[:, D + DH:] = prm_ref[1]


_tc_call = pl.pallas_call(
    _tc_body,
    grid=(N_PAD // TBLK,),
    in_specs=[
        pl.BlockSpec((NC, TBLK, D), lambda i: (0, i, 0)),
        pl.BlockSpec((NC, TBLK, DH), lambda i: (0, i, 0)),
    ],
    out_specs=pl.BlockSpec((TBLK, 2 * D), lambda i: (i, 0)),
    out_shape=jax.ShapeDtypeStruct((N_PAD, 2 * D), jnp.float32),
)


@jax.jit
def kernel(graph_embedding, edge_index, weight):
    src = edge_index[0].astype(jnp.int32)
    dst = edge_index[1].astype(jnp.int32)
    pad = E_PAD - E
    src = jnp.concatenate([src, jnp.zeros((pad,), jnp.int32)])
    dst = jnp.concatenate([dst, jnp.full((pad,), N, jnp.int32)])
    srcg = src.reshape(NC, NS, CHUNKS, CH)
    dstg = dst.reshape(NC, NS, CHUNKS, CH)
    w3 = weight.reshape(NC, 1, DH)
    acc_parts, prm_parts = _sc_call(graph_embedding, srcg, dstg, w3)
    return _tc_call(acc_parts, prm_parts)[:N]


# speedup vs baseline: 4.3041x; 1.1294x over previous
import functools

import jax
import jax.numpy as jnp
from jax import lax
from jax.experimental import pallas as pl
from jax.experimental.pallas import tpu as pltpu
from jax.experimental.pallas import tpu_sc as plsc

N = 10000
E = 320000
D = 128
DH = 64
NC = 2
NS = 16
CH = 64
PCH = 64
IB = 16
NBLK = 10
CHUNKS = IB * NBLK
E_PAD = CHUNKS * NC * NS * CH
N_PAD = 10240
ZR = N_PAD // NS
DR = N_PAD // D
TBLK = 1024


def _sc_body(emb_hbm, src_hbm, dst_hbm, w_hbm, acc_out, prm_out,
             acc, deg, hist, sidx, didx, didx2, gbuf, pstage, dbuf, wv, iota,
             gsem, ssem):
    c = lax.axis_index("c")
    s = lax.axis_index("s")
    r0 = s * ZR

    zf = jnp.zeros((16,), jnp.float32)

    def zrow(r, carry):
        for k in range(D // 16):
            gbuf[0, r, pl.ds(k * 16, 16)] = zf
        return carry

    lax.fori_loop(0, CH, zrow, 0)

    def zh(i, carry):
        for k in range(D // 16):
            hist[i, pl.ds(k * 16, 16)] = zf
        return carry

    lax.fori_loop(0, DR, zh, 0)

    for b in range(ZR // CH):
        pltpu.sync_copy(gbuf.at[0], acc.at[pl.ds(r0 + b * CH, CH), :])

    @pl.when(s == 0)
    def _():
        pltpu.sync_copy(gbuf.at[0, pl.ds(0, DR), :], deg)

    it16 = lax.iota(jnp.int32, 16)
    for k in range(DR // 16):
        iota[0, pl.ds(k * 16, 16)] = it16 + 16 * k

    pltpu.sync_copy(w_hbm.at[c], wv)

    plsc.subcore_barrier()

    ones16 = jnp.ones((16,), jnp.float32)

    def hgroup(idxvec):
        plsc.addupdate_scatter(
            hist,
            [lax.shift_right_logical(idxvec, 7),
             lax.bitwise_and(idxvec, D - 1)],
            ones16)

    def blk(b, carry):
        pltpu.sync_copy(src_hbm.at[c, s, pl.ds(b * IB, IB), :], sidx)
        pltpu.sync_copy(dst_hbm.at[c, s, pl.ds(b * IB, IB), :], didx)
        pltpu.sync_copy(dst_hbm.at[1 - c, s, pl.ds(b * IB, IB), :], didx2)

        gs = [None, None]
        ss = [None, None]
        for j in range(IB):
            p = j & 1
            if ss[p] is not None:
                ss[p].wait()
            gs[p] = pltpu.async_copy(emb_hbm.at[sidx.at[j]], gbuf.at[p],
                                     gsem[p])
            for k in range(CH // 16):
                hgroup(didx[j, pl.ds(k * 16, 16)])
                hgroup(didx2[j, pl.ds(k * 16, 16)])
            if j > 0:
                q = (j - 1) & 1
                gs[q].wait()
                ss[q] = pltpu.async_copy(gbuf.at[q], acc.at[didx.at[j - 1]],
                                         ssem[q], add=True)
        q = (IB - 1) & 1
        gs[q].wait()
        ss[q] = pltpu.async_copy(gbuf.at[q], acc.at[didx.at[IB - 1]],
                                 ssem[q], add=True)
        ss[0].wait()
        ss[1].wait()
        return carry

    lax.fori_loop(0, NBLK, blk, 0)

    pltpu.sync_copy(hist, deg.at[iota.at[0]], add=True)

    plsc.subcore_barrier()

    for b in range(ZR // CH):
        pltpu.sync_copy(acc.at[pl.ds(r0 + b * CH, CH), :], gbuf.at[b & 1])
        pltpu.sync_copy(gbuf.at[b & 1],
                        acc_out.at[c, pl.ds(r0 + b * CH, CH), :])

    pltpu.sync_copy(deg.at[pl.ds(s * (ZR // D), ZR // D), :], dbuf)
    wvecs = [wv[0, pl.ds(k * 16, 16)] for k in range(DH // 16)]

    for bb in range(ZR // PCH):
        def prow(j, carry, bb=bb):
            row = bb * PCH + j
            dl = plsc.load_gather(
                dbuf, [jnp.full((16,), row // D, jnp.int32),
                       jnp.full((16,), row % D, jnp.int32)])
            for k in range(DH // 16):
                pstage[j, pl.ds(k * 16, 16)] = dl * wvecs[k]
            return carry

        lax.fori_loop(0, PCH, prow, 0)
        pltpu.sync_copy(pstage, prm_out.at[c, pl.ds(r0 + bb * PCH, PCH), :])


_sc_call = pl.kernel(
    _sc_body,
    out_type=(
        jax.ShapeDtypeStruct((NC, N_PAD, D), jnp.float32),
        jax.ShapeDtypeStruct((NC, N_PAD, DH), jnp.float32),
    ),
    mesh=plsc.VectorSubcoreMesh(core_axis_name="c", subcore_axis_name="s"),
    compiler_params=pltpu.CompilerParams(needs_layout_passes=False),
    scratch_types=[
        pltpu.VMEM_SHARED((N_PAD, D), jnp.float32),
        pltpu.VMEM_SHARED((DR, D), jnp.float32),
        pltpu.VMEM((DR, D), jnp.float32),
        pltpu.VMEM((IB, CH), jnp.int32),
        pltpu.VMEM((IB, CH), jnp.int32),
        pltpu.VMEM((IB, CH), jnp.int32),
        pltpu.VMEM((2, CH, D), jnp.float32),
        pltpu.VMEM((PCH, DH), jnp.float32),
        pltpu.VMEM((ZR // D, D), jnp.float32),
        pltpu.VMEM((1, DH), jnp.float32),
        pltpu.VMEM((1, DR), jnp.int32),
        (pltpu.SemaphoreType.DMA, pltpu.SemaphoreType.DMA),
        (pltpu.SemaphoreType.DMA, pltpu.SemaphoreType.DMA),
    ],
)


def _tc_body(acc_ref, prm_ref, out_ref):
    out_ref[:, :D] = acc_ref[0] + acc_ref[1]
    out_ref[:, D:D + DH] = prm_ref[0]
    out_ref[:, D + DH:] = prm_ref[1]


_tc_call = pl.pallas_call(
    _tc_body,
    grid=(N_PAD // TBLK,),
    in_specs=[
        pl.BlockSpec((NC, TBLK, D), lambda i: (0, i, 0)),
        pl.BlockSpec((NC, TBLK, DH), lambda i: (0, i, 0)),
    ],
    out_specs=pl.BlockSpec((TBLK, 2 * D), lambda i: (i, 0)),
    out_shape=jax.ShapeDtypeStruct((N_PAD, 2 * D), jnp.float32),
)


@jax.jit
def kernel(graph_embedding, edge_index, weight):
    src = edge_index[0].astype(jnp.int32)
    dst = edge_index[1].astype(jnp.int32)
    pad = E_PAD - E
    src = jnp.concatenate([src, jnp.zeros((pad,), jnp.int32)])
    dst = jnp.concatenate([dst, jnp.full((pad,), N, jnp.int32)])
    srcg = src.reshape(NC, NS, CHUNKS, CH)
    dstg = dst.reshape(NC, NS, CHUNKS, CH)
    w3 = weight.reshape(NC, 1, DH)
    acc_parts, prm_parts = _sc_call(graph_embedding, srcg, dstg, w3)
    return _tc_call(acc_parts, prm_parts)[:N]
